# baseline (speedup 1.0000x reference)
import jax
import jax.numpy as jnp
from jax.experimental import pallas as pl
from jax.experimental.pallas import tpu as pltpu

B = 64
VISUAL_DIM = 512
HW = 256
CODE_DIM = 64
NUM_CODES = 1024
HIDDEN_DIM = 256
MAX_STEPS = 8
COMMITMENT_COST = 0.25
N_BLK = 8
BB = B // N_BLK
N_BUF = 3


def _dot_t(x, w):
    return jax.lax.dot_general(x, w, (((1,), (1,)), ((), ())),
                               preferred_element_type=jnp.float32)


def _body(spat_ref, bos_ref, wih_ref, whh_ref, bih_ref, bhh_ref,
          gw_ref, gb_ref, bw_ref, bb_ref, w1_ref, b1_ref,
          w2_ref, b2_ref, cb_ref,
          h_out, sel_out, idx_out, z_out, vq_out,
          buf, sem, mean_scr):
    for k in range(N_BUF):
        pltpu.make_async_copy(spat_ref.at[pl.ds(k * BB, BB)],
                              buf.at[k], sem.at[k]).start()
    for i in range(N_BLK):
        k = i % N_BUF
        pltpu.make_async_copy(spat_ref.at[pl.ds(i * BB, BB)],
                              buf.at[k], sem.at[k]).wait()
        mean_scr[pl.ds(i * BB, BB), :] = jnp.sum(buf[k], axis=2) * (1.0 / HW)
        nxt = i + N_BUF
        if nxt < N_BLK:
            pltpu.make_async_copy(spat_ref.at[pl.ds(nxt * BB, BB)],
                                  buf.at[k], sem.at[k]).start()

    sp_mean = mean_scr[...]
    wih = wih_ref[...]
    whh = whh_ref[...]
    bih = bih_ref[...]
    bhh = bhh_ref[...]
    gw = gw_ref[...]
    gb = gb_ref[...]
    bw = bw_ref[...]
    bb_ = bb_ref[...]
    w1 = w1_ref[...]
    b1 = b1_ref[...]
    w2 = w2_ref[...]
    b2 = b2_ref[...]
    cb = cb_ref[...]
    c2 = jnp.sum(cb * cb, axis=1)[None, :]
    w1h = w1[:, :HIDDEN_DIM]
    w1c = w1[:, HIDDEN_DIM:]

    h = jnp.zeros((B, HIDDEN_DIM), jnp.float32)
    prev = jnp.broadcast_to(bos_ref[...], (B, CODE_DIM))
    total_vq = jnp.float32(0.0)
    for t in range(MAX_STEPS):
        gi = _dot_t(prev, wih) + bih
        gh = _dot_t(h, whh) + bhh
        r = jax.nn.sigmoid(gi[:, :HIDDEN_DIM] + gh[:, :HIDDEN_DIM])
        z = jax.nn.sigmoid(gi[:, HIDDEN_DIM:2 * HIDDEN_DIM]
                           + gh[:, HIDDEN_DIM:2 * HIDDEN_DIM])
        n = jnp.tanh(gi[:, 2 * HIDDEN_DIM:] + r * gh[:, 2 * HIDDEN_DIM:])
        h = (1.0 - z) * n + z * h
        gamma = _dot_t(h, gw) + gb
        beta = _dot_t(h, bw) + bb_
        c_t = (1.0 + gamma) * sp_mean + beta
        hid = _dot_t(h, w1h) + _dot_t(c_t, w1c) + b1
        hid = jnp.maximum(hid, 0.0)
        z_cont = _dot_t(hid, w2) + b2
        z2 = jnp.sum(z_cont * z_cont, axis=1, keepdims=True)
        zc = _dot_t(z_cont, cb)
        d = z2 - 2.0 * zc + c2
        idx = jnp.argmin(d, axis=1).astype(jnp.int32)
        onehot = (jax.lax.broadcasted_iota(jnp.int32, (B, NUM_CODES), 1)
                  == idx[:, None]).astype(jnp.float32)
        z_q = jnp.dot(onehot, cb, preferred_element_type=jnp.float32)
        diff = z_q - z_cont
        total_vq = total_vq + jnp.sum(diff * diff)
        sel_out[:, t, :] = z_q
        idx_out[:, t] = idx
        z_out[:, t, :] = z_cont
        prev = z_q
    h_out[...] = h
    scale = COMMITMENT_COST / (MAX_STEPS * B * CODE_DIM)
    vq_out[...] = jnp.full((1, 1), scale) * total_vq


def kernel(spatial_features, bos_token, gru_w_ih, gru_w_hh, gru_b_ih, gru_b_hh,
           gamma_w, gamma_b, beta_w, beta_b, probe_w1, probe_b1, probe_w2,
           probe_b2, codebook):
    operands = (
        spatial_features.reshape(B, VISUAL_DIM, HW),
        bos_token.reshape(1, CODE_DIM),
        gru_w_ih,
        gru_w_hh,
        gru_b_ih.reshape(1, -1),
        gru_b_hh.reshape(1, -1),
        gamma_w,
        gamma_b.reshape(1, -1),
        beta_w,
        beta_b.reshape(1, -1),
        probe_w1,
        probe_b1.reshape(1, -1),
        probe_w2,
        probe_b2.reshape(1, -1),
        codebook,
    )

    in_specs = [pl.BlockSpec(memory_space=pltpu.MemorySpace.HBM)]
    in_specs += [pl.BlockSpec(memory_space=pltpu.MemorySpace.VMEM)
                 for _ in operands[1:]]

    out_shapes = (
        jax.ShapeDtypeStruct((B, HIDDEN_DIM), jnp.float32),
        jax.ShapeDtypeStruct((B, MAX_STEPS, CODE_DIM), jnp.float32),
        jax.ShapeDtypeStruct((B, MAX_STEPS), jnp.int32),
        jax.ShapeDtypeStruct((B, MAX_STEPS, CODE_DIM), jnp.float32),
        jax.ShapeDtypeStruct((1, 1), jnp.float32),
    )

    h, sel, idx, zc, vq = pl.pallas_call(
        _body,
        out_shape=out_shapes,
        scratch_shapes=[
            pltpu.VMEM((N_BUF, BB, VISUAL_DIM, HW), jnp.float32),
            pltpu.SemaphoreType.DMA((N_BUF,)),
            pltpu.VMEM((B, VISUAL_DIM), jnp.float32),
        ],
    )(*operands)

    return (h, sel, idx, zc, vq.reshape(()))


# speedup vs baseline: 1.1232x; 1.1232x over previous
import jax
import jax.numpy as jnp
from jax.experimental import pallas as pl
from jax.experimental.pallas import tpu as pltpu

B = 64
VISUAL_DIM = 512
HW = 256
CODE_DIM = 64
NUM_CODES = 1024
HIDDEN_DIM = 256
MAX_STEPS = 8
COMMITMENT_COST = 0.25
N_BLK = 8
BB = B // N_BLK


def _dot_t(x, w):
    return jax.lax.dot_general(x, w, (((1,), (1,)), ((), ())),
                               preferred_element_type=jnp.float32)


N_BUF = 4


def _mean_body(spat_ref, out_ref, buf, sem):
    for k in range(N_BUF):
        pltpu.make_async_copy(spat_ref.at[pl.ds(k * BB, BB)],
                              buf.at[k], sem.at[k]).start()
    for i in range(N_BLK):
        k = i % N_BUF
        pltpu.make_async_copy(spat_ref.at[pl.ds(i * BB, BB)],
                              buf.at[k], sem.at[k]).wait()
        out_ref[pl.ds(i * BB, BB), :] = jnp.sum(buf[k], axis=2) * (1.0 / HW)
        nxt = i + N_BUF
        if nxt < N_BLK:
            pltpu.make_async_copy(spat_ref.at[pl.ds(nxt * BB, BB)],
                                  buf.at[k], sem.at[k]).start()


def _rec_body(mean_ref, bos_ref, wih_ref, whh_ref, bih_ref, bhh_ref,
              gw_ref, gb_ref, bw_ref, bb_ref, w1_ref, b1_ref,
              w2_ref, b2_ref, cb_ref,
              h_out, sel_out, idx_out, z_out, vq_out):
    sp_mean = mean_ref[...]
    wih = wih_ref[...]
    whh = whh_ref[...]
    bih = bih_ref[...]
    bhh = bhh_ref[...]
    gw = gw_ref[...]
    gb = gb_ref[...]
    bw = bw_ref[...]
    bb_ = bb_ref[...]
    w1 = w1_ref[...]
    b1 = b1_ref[...]
    w2 = w2_ref[...]
    b2 = b2_ref[...]
    cb = cb_ref[...]
    c2 = jnp.sum(cb * cb, axis=1)[None, :]
    w1h = w1[:, :HIDDEN_DIM]
    w1c = w1[:, HIDDEN_DIM:]

    h = jnp.zeros((B, HIDDEN_DIM), jnp.float32)
    prev = jnp.broadcast_to(bos_ref[...], (B, CODE_DIM))
    total_vq = jnp.float32(0.0)
    for t in range(MAX_STEPS):
        gi = _dot_t(prev, wih) + bih
        gh = _dot_t(h, whh) + bhh
        r = jax.nn.sigmoid(gi[:, :HIDDEN_DIM] + gh[:, :HIDDEN_DIM])
        z = jax.nn.sigmoid(gi[:, HIDDEN_DIM:2 * HIDDEN_DIM]
                           + gh[:, HIDDEN_DIM:2 * HIDDEN_DIM])
        n = jnp.tanh(gi[:, 2 * HIDDEN_DIM:] + r * gh[:, 2 * HIDDEN_DIM:])
        h = (1.0 - z) * n + z * h
        gamma = _dot_t(h, gw) + gb
        beta = _dot_t(h, bw) + bb_
        c_t = (1.0 + gamma) * sp_mean + beta
        hid = _dot_t(h, w1h) + _dot_t(c_t, w1c) + b1
        hid = jnp.maximum(hid, 0.0)
        z_cont = _dot_t(hid, w2) + b2
        z2 = jnp.sum(z_cont * z_cont, axis=1, keepdims=True)
        zc = _dot_t(z_cont, cb)
        d = z2 - 2.0 * zc + c2
        idx = jnp.argmin(d, axis=1).astype(jnp.int32)
        onehot = (jax.lax.broadcasted_iota(jnp.int32, (B, NUM_CODES), 1)
                  == idx[:, None]).astype(jnp.float32)
        z_q = jnp.dot(onehot, cb, preferred_element_type=jnp.float32)
        diff = z_q - z_cont
        total_vq = total_vq + jnp.sum(diff * diff)
        sel_out[:, t, :] = z_q
        idx_out[:, t] = idx
        z_out[:, t, :] = z_cont
        prev = z_q
    h_out[...] = h
    scale = COMMITMENT_COST / (MAX_STEPS * B * CODE_DIM)
    vq_out[...] = jnp.full((1, 1), scale) * total_vq


def kernel(spatial_features, bos_token, gru_w_ih, gru_w_hh, gru_b_ih, gru_b_hh,
           gamma_w, gamma_b, beta_w, beta_b, probe_w1, probe_b1, probe_w2,
           probe_b2, codebook):
    spat3 = spatial_features.reshape(B, VISUAL_DIM, HW)

    sp_mean = pl.pallas_call(
        _mean_body,
        in_specs=[pl.BlockSpec(memory_space=pltpu.MemorySpace.HBM)],
        out_shape=jax.ShapeDtypeStruct((B, VISUAL_DIM), jnp.float32),
        scratch_shapes=[
            pltpu.VMEM((N_BUF, BB, VISUAL_DIM, HW), jnp.float32),
            pltpu.SemaphoreType.DMA((N_BUF,)),
        ],
    )(spat3)

    operands = (
        sp_mean,
        bos_token.reshape(1, CODE_DIM),
        gru_w_ih,
        gru_w_hh,
        gru_b_ih.reshape(1, -1),
        gru_b_hh.reshape(1, -1),
        gamma_w,
        gamma_b.reshape(1, -1),
        beta_w,
        beta_b.reshape(1, -1),
        probe_w1,
        probe_b1.reshape(1, -1),
        probe_w2,
        probe_b2.reshape(1, -1),
        codebook,
    )

    out_shapes = (
        jax.ShapeDtypeStruct((B, HIDDEN_DIM), jnp.float32),
        jax.ShapeDtypeStruct((B, MAX_STEPS, CODE_DIM), jnp.float32),
        jax.ShapeDtypeStruct((B, MAX_STEPS), jnp.int32),
        jax.ShapeDtypeStruct((B, MAX_STEPS, CODE_DIM), jnp.float32),
        jax.ShapeDtypeStruct((1, 1), jnp.float32),
    )

    h, sel, idx, zc, vq = pl.pallas_call(
        _rec_body,
        out_shape=out_shapes,
    )(*operands)

    return (h, sel, idx, zc, vq.reshape(()))
